# baseline (speedup 1.0000x reference)
import jax
import jax.numpy as jnp
from jax import lax
from jax.experimental import pallas as pl
from jax.experimental.pallas import tpu as pltpu
from jax.experimental.pallas import tpu_sc as plsc

VOCAB = 1000000
EMBED = 32
BATCH = 4096
SEQ = 200

_NC, _NS = 2, 16
_NW = _NC * _NS
_TB = BATCH // 128
_NBLK = SEQ * _TB
_BPW = _NBLK // _NW
_NBUF = 3
_NOUTER = (_BPW + 2 + _NBUF - 1) // _NBUF


def _sc_body(xt_hbm, tok_hbm, pos_hbm, out_hbm,
             idx_v, buf0, buf1, buf2, obuf, sem_g, sem_w):
    bufs = (buf0, buf1, buf2)
    wid = lax.axis_index("s") * _NC + lax.axis_index("c")
    g0 = wid * _BPW

    def tok_copy(b):
        return pltpu.make_async_copy(tok_hbm.at[idx_v.at[b]], bufs[b],
                                     sem_g.at[b])

    def pos_copy(b):
        return pltpu.make_async_copy(pos_hbm.at[idx_v.at[b]], bufs[b],
                                     sem_g.at[b])

    def out_copy(g, b):
        blk = g0 + g
        s = lax.div(blk, _TB)
        tb = lax.rem(blk, _TB)
        return pltpu.make_async_copy(obuf.at[b], out_hbm.at[s, :, tb],
                                     sem_w.at[b])

    lanes = lax.iota(jnp.int32, 16)

    def step(ci, _):
        for k in range(_NBUF):
            g = ci * _NBUF + k
            b0 = k
            b1 = (k - 1) % _NBUF
            b2 = (k - 2) % _NBUF

            @pl.when(g < _BPW)
            def _stage_a(g=g, b0=b0):
                blk = g0 + g
                s_a = lax.div(blk, _TB)
                tb_a = lax.rem(blk, _TB)

                @pl.when(g >= _NBUF)
                def _reclaim():
                    out_copy(g - _NBUF, b0).wait()
                pltpu.sync_copy(xt_hbm.at[s_a, tb_a], idx_v.at[b0])
                tok_copy(b0).start()

            @pl.when(jnp.logical_and(g >= 1, g - 1 < _BPW))
            def _stage_b(b1=b1):
                tok_copy(b1).wait()
                pltpu.async_copy(pos_hbm.at[idx_v.at[b1]], bufs[b1],
                                 sem_g.at[b1], add=True)

            @pl.when(jnp.logical_and(g >= 2, g - 2 < _BPW))
            def _stage_c(g=g, b2=b2):
                pos_copy(b2).wait()
                for c in range(8):
                    b16 = c * 16 + lanes
                    for e in range(EMBED):
                        e16 = jnp.full((16,), e, jnp.int32)
                        col = plsc.load_gather(bufs[b2], [b16, e16])
                        obuf[b2, e // 8, e % 8, pl.ds(c * 16, 16)] = col
                out_copy(g - 2, b2).start()

        return 0

    lax.fori_loop(0, _NOUTER, step, 0)
    for k in range(_NBUF):
        gg = _BPW - _NBUF + k
        out_copy(gg, gg % _NBUF).wait()


@jax.jit
def kernel(x, token_table, pos_table):
    xt = x.T.reshape(SEQ, _TB, 128)
    mesh = plsc.VectorSubcoreMesh(core_axis_name="c", subcore_axis_name="s")
    P = pl.kernel(
        _sc_body,
        out_type=jax.ShapeDtypeStruct((SEQ, EMBED // 8, _TB, 8, 128),
                                      jnp.float32),
        mesh=mesh,
        scratch_types=[
            pltpu.VMEM((_NBUF, 128), jnp.int32),
            pltpu.VMEM((128, EMBED), jnp.float32),
            pltpu.VMEM((128, EMBED), jnp.float32),
            pltpu.VMEM((128, EMBED), jnp.float32),
            pltpu.VMEM((_NBUF, EMBED // 8, 8, 128), jnp.float32),
            pltpu.SemaphoreType.DMA((_NBUF,)),
            pltpu.SemaphoreType.DMA((_NBUF,)),
        ],
        compiler_params=pltpu.CompilerParams(use_tc_tiling_on_sc=False,
                                             needs_layout_passes=False),
    )(xt, token_table, pos_table)
    return P.transpose(2, 4, 0, 1, 3).reshape(BATCH, SEQ, EMBED)


# speedup vs baseline: 1.0204x; 1.0204x over previous
import jax
import jax.numpy as jnp
from jax import lax
from jax.experimental import pallas as pl
from jax.experimental.pallas import tpu as pltpu
from jax.experimental.pallas import tpu_sc as plsc

VOCAB = 1000000
EMBED = 32
BATCH = 4096
SEQ = 200

_NC, _NS = 2, 16
_NW = _NC * _NS
_TB = BATCH // 128
_JU = 2
_QU = _TB // _JU
_NUNIT = SEQ * _QU
_UPW = _NUNIT // _NW
_NBUF = 3
_NOUTER = (_UPW + 2 + _NBUF - 1) // _NBUF


def _sc_body(xt_hbm, tok_hbm, pos_hbm, out_hbm,
             idx_v, qidx_v, buf0, buf1, buf2, obuf, sem_g, sem_w):
    bufs = (buf0, buf1, buf2)
    wid = lax.axis_index("s") * _NC + lax.axis_index("c")
    g0 = wid * _UPW

    def tok_copy(b, j):
        return pltpu.make_async_copy(
            tok_hbm.at[qidx_v.at[b, j]],
            bufs[b].at[pl.ds(j * 128, 128)], sem_g.at[b])

    def pos_copy(b, j):
        return pltpu.make_async_copy(
            pos_hbm.at[qidx_v.at[b, j]],
            bufs[b].at[pl.ds(j * 128, 128)], sem_g.at[b])

    def out_copy(g, b):
        u = g0 + g
        s = lax.div(u, _QU)
        q = lax.rem(u, _QU)
        return pltpu.make_async_copy(
            obuf.at[b], out_hbm.at[s, :, pl.ds(q * _JU, _JU)], sem_w.at[b])

    def step(ci, _):
        lanes = lax.iota(jnp.int32, 16)
        for k in range(_NBUF):
            g = ci * _NBUF + k
            b0 = k
            b1 = (k - 1) % _NBUF
            b2 = (k - 2) % _NBUF

            @pl.when(g < _UPW)
            def _stage_a(g=g, b0=b0):
                u = g0 + g
                s_a = lax.div(u, _QU)
                q_a = lax.rem(u, _QU)

                @pl.when(g >= _NBUF)
                def _reclaim():
                    out_copy(g - _NBUF, b0).wait()
                pltpu.sync_copy(xt_hbm.at[s_a, pl.ds(q_a * _JU, _JU)],
                                idx_v.at[b0])
                for j in range(_JU):
                    for c in range(8):
                        v = idx_v[b0, j, pl.ds(c * 16, 16)]
                        qidx_v[b0, j, pl.ds(c * 16, 16)] = v >> 2
                for j in range(_JU):
                    tok_copy(b0, j).start()

            @pl.when(jnp.logical_and(g >= 1, g - 1 < _UPW))
            def _stage_b(b1=b1):
                for j in range(_JU):
                    tok_copy(b1, j).wait()
                for j in range(_JU):
                    pltpu.async_copy(pos_hbm.at[qidx_v.at[b1, j]],
                                     bufs[b1].at[pl.ds(j * 128, 128)],
                                     sem_g.at[b1], add=True)

            @pl.when(jnp.logical_and(g >= 2, g - 2 < _UPW))
            def _stage_c(g=g, b2=b2):
                for j in range(_JU):
                    pos_copy(b2, j).wait()

                def xpose(jc, _c):
                    j = lax.div(jc, 8)
                    c = lax.rem(jc, 8)
                    idxc = idx_v[b2, j, pl.ds(c * 16, 16)]
                    off16 = (idxc & 3) * 32
                    b16 = j * 128 + c * 16 + lanes
                    for e in range(EMBED):
                        col = plsc.load_gather(bufs[b2], [b16, off16 + e])
                        obuf[b2, e // 8, j, e % 8, pl.ds(c * 16, 16)] = col
                    return 0

                lax.fori_loop(0, _JU * 8, xpose, 0)
                out_copy(g - 2, b2).start()

        return 0

    lax.fori_loop(0, _NOUTER, step, 0)
    for k in range(_NBUF):
        gg = _UPW - _NBUF + k
        out_copy(gg, gg % _NBUF).wait()


@jax.jit
def kernel(x, token_table, pos_table):
    xt = x.T.reshape(SEQ, _TB, 128)
    tq = token_table.reshape(VOCAB // 4, 128)
    pq = pos_table.reshape(VOCAB // 4, 128)
    mesh = plsc.VectorSubcoreMesh(core_axis_name="c", subcore_axis_name="s")
    P = pl.kernel(
        _sc_body,
        out_type=jax.ShapeDtypeStruct((SEQ, EMBED // 8, _TB, 8, 128),
                                      jnp.float32),
        mesh=mesh,
        scratch_types=[
            pltpu.VMEM((_NBUF, _JU, 128), jnp.int32),
            pltpu.VMEM((_NBUF, _JU, 128), jnp.int32),
            pltpu.VMEM((_JU * 128, 128), jnp.float32),
            pltpu.VMEM((_JU * 128, 128), jnp.float32),
            pltpu.VMEM((_JU * 128, 128), jnp.float32),
            pltpu.VMEM((_NBUF, EMBED // 8, _JU, 8, 128), jnp.float32),
            pltpu.SemaphoreType.DMA((_NBUF,)),
            pltpu.SemaphoreType.DMA((_NBUF,)),
        ],
        compiler_params=pltpu.CompilerParams(use_tc_tiling_on_sc=True,
                                             needs_layout_passes=False),
    )(xt, tq, pq)
    return P.transpose(2, 4, 0, 1, 3).reshape(BATCH, SEQ, EMBED)


# speedup vs baseline: 1.5431x; 1.5123x over previous
import jax
import jax.numpy as jnp
from jax import lax
from jax.experimental import pallas as pl
from jax.experimental.pallas import tpu as pltpu
from jax.experimental.pallas import tpu_sc as plsc

VOCAB = 1000000
EMBED = 32
BATCH = 4096
SEQ = 200

_NC, _NS = 2, 16
_NW = _NC * _NS
_TB = BATCH // 128
_JU = 4
_QU = _TB // _JU
_NUNIT = SEQ * _QU
_UPW = _NUNIT // _NW
_ROWS = _JU * 128
_NBUF = 3
_NOUTER = (_UPW + 2 + _NBUF - 1) // _NBUF
_PB = 129


def _sc_body(xt_hbm, tok_hbm, pos_hbm, out_hbm,
             idx_v, buf0, buf1, buf2, ob0, ob1, ob2, sem_g, sem_w):
    bufs = (buf0, buf1, buf2)
    obufs = (ob0, ob1, ob2)
    wid = lax.axis_index("s") * _NC + lax.axis_index("c")
    g0 = wid * _UPW

    def tok_copy(b, j):
        return pltpu.make_async_copy(
            tok_hbm.at[idx_v.at[b, j]],
            bufs[b].at[pl.ds(j * 128, 128)], sem_g.at[b])

    def pos_copy(b, j):
        return pltpu.make_async_copy(
            pos_hbm.at[idx_v.at[b, j]],
            bufs[b].at[pl.ds(j * 128, 128)], sem_g.at[b])

    def out_copies(g, b):
        u = g0 + g
        s = lax.div(u, _QU)
        q = lax.rem(u, _QU)
        return [pltpu.make_async_copy(
                    obufs[b].at[pl.ds(te * (_JU * 8), _JU * 8), pl.ds(0, 128)],
                    out_hbm.at[s, te, pl.ds(q * (_JU * 8), _JU * 8)],
                    sem_w.at[b])
                for te in range(4)]

    def step(ci, _):
        lanes = lax.iota(jnp.int32, 16)
        rv0 = (lanes // 8) * (_JU * 8) + (lanes % 8)
        rv1 = rv0 + 2 * (_JU * 8)
        for k in range(_NBUF):
            g = ci * _NBUF + k
            b0 = k
            b1 = (k - 1) % _NBUF
            b2 = (k - 2) % _NBUF

            @pl.when(g < _UPW)
            def _stage_a(g=g, b0=b0):
                u = g0 + g
                s_a = lax.div(u, _QU)
                q_a = lax.rem(u, _QU)

                @pl.when(g >= _NBUF)
                def _reclaim():
                    for cp in out_copies(g - _NBUF, b0):
                        cp.wait()
                pltpu.sync_copy(xt_hbm.at[s_a, pl.ds(q_a * _JU, _JU)],
                                idx_v.at[b0])
                for j in range(_JU):
                    tok_copy(b0, j).start()

            @pl.when(jnp.logical_and(g >= 1, g - 1 < _UPW))
            def _stage_b(b1=b1):
                for j in range(_JU):
                    tok_copy(b1, j).wait()
                for j in range(_JU):
                    pltpu.async_copy(pos_hbm.at[idx_v.at[b1, j]],
                                     bufs[b1].at[pl.ds(j * 128, 128)],
                                     sem_g.at[b1], add=True)

            @pl.when(jnp.logical_and(g >= 2, g - 2 < _UPW))
            def _stage_c(g=g, b2=b2):
                for j in range(_JU):
                    pos_copy(b2, j).wait()

                def xpose(it, _c):
                    row0 = it * 8
                    j = lax.div(it, 16)
                    rj0 = rv0 + j * 8
                    rj1 = rv1 + j * 8
                    col = row0 - j * 128
                    for r in range(8):
                        v0 = bufs[b2][row0 + r, pl.ds(0, 16)]
                        v1 = bufs[b2][row0 + r, pl.ds(16, 16)]
                        cvec = jnp.full((16,), col + r, jnp.int32)
                        plsc.store_scatter(obufs[b2], [rj0, cvec], v0)
                        plsc.store_scatter(obufs[b2], [rj1, cvec], v1)
                    return 0

                lax.fori_loop(0, _ROWS // 8, xpose, 0)
                for cp in out_copies(g - 2, b2):
                    cp.start()

        return 0

    lax.fori_loop(0, _NOUTER, step, 0)
    for k in range(_NBUF):
        gg = _UPW - _NBUF + k
        for cp in out_copies(gg, gg % _NBUF):
            cp.wait()


@jax.jit
def kernel(x, token_table, pos_table):
    xt = x.T.reshape(SEQ, _TB, 128)
    mesh = plsc.VectorSubcoreMesh(core_axis_name="c", subcore_axis_name="s")
    P = pl.kernel(
        _sc_body,
        out_type=jax.ShapeDtypeStruct((SEQ, EMBED // 8, _TB * 8, 128),
                                      jnp.float32),
        mesh=mesh,
        scratch_types=[
            pltpu.VMEM((_NBUF, _JU, 128), jnp.int32),
            pltpu.VMEM((_ROWS, EMBED), jnp.float32),
            pltpu.VMEM((_ROWS, EMBED), jnp.float32),
            pltpu.VMEM((_ROWS, EMBED), jnp.float32),
            pltpu.VMEM((4 * _JU * 8, _PB), jnp.float32),
            pltpu.VMEM((4 * _JU * 8, _PB), jnp.float32),
            pltpu.VMEM((4 * _JU * 8, _PB), jnp.float32),
            pltpu.SemaphoreType.DMA((_NBUF,)),
            pltpu.SemaphoreType.DMA((_NBUF,)),
        ],
        compiler_params=pltpu.CompilerParams(use_tc_tiling_on_sc=False,
                                             needs_layout_passes=False),
    )(xt, token_table, pos_table)
    return (P.reshape(SEQ, EMBED // 8, _TB, 8, 128)
             .transpose(2, 4, 0, 1, 3).reshape(BATCH, SEQ, EMBED))


# speedup vs baseline: 1.5479x; 1.0031x over previous
import jax
import jax.numpy as jnp
from jax import lax
from jax.experimental import pallas as pl
from jax.experimental.pallas import tpu as pltpu
from jax.experimental.pallas import tpu_sc as plsc

VOCAB = 1000000
EMBED = 32
BATCH = 4096
SEQ = 200

_NC, _NS = 2, 16
_NW = _NC * _NS
_TB = BATCH // 128
_JU = 4
_QU = _TB // _JU
_NUNIT = SEQ * _QU
_UPW = _NUNIT // _NW
_ROWS = _JU * 128
_NBUF = 3
_NOUTER = (_UPW + 2 + _NBUF - 1) // _NBUF
_PB = 129


def _sc_body(xt_hbm, tok_hbm, pos_hbm, out_hbm,
             idx_v, buf0, buf1, buf2, ob0, ob1, ob2, sem_g, sem_w, sem_i):
    bufs = (buf0, buf1, buf2)
    obufs = (ob0, ob1, ob2)
    wid = lax.axis_index("s") * _NC + lax.axis_index("c")
    g0 = wid * _UPW

    def idx_copy(g, b):
        u = g0 + g
        s = lax.div(u, _QU)
        q = lax.rem(u, _QU)
        return pltpu.make_async_copy(
            xt_hbm.at[s, pl.ds(q * _JU, _JU)], idx_v.at[b], sem_i.at[b])

    def tok_copy(b, j):
        return pltpu.make_async_copy(
            tok_hbm.at[idx_v.at[b, j]],
            bufs[b].at[pl.ds(j * 128, 128)], sem_g.at[b])

    def pos_copy(b, j):
        return pltpu.make_async_copy(
            pos_hbm.at[idx_v.at[b, j]],
            bufs[b].at[pl.ds(j * 128, 128)], sem_g.at[b])

    def out_copies(g, b):
        u = g0 + g
        s = lax.div(u, _QU)
        q = lax.rem(u, _QU)
        return [pltpu.make_async_copy(
                    obufs[b].at[pl.ds(te * (_JU * 8), _JU * 8), pl.ds(0, 128)],
                    out_hbm.at[s, te, pl.ds(q * (_JU * 8), _JU * 8)],
                    sem_w.at[b])
                for te in range(4)]

    def step(ci, _):
        lanes = lax.iota(jnp.int32, 16)
        rv0 = (lanes // 8) * (_JU * 8) + (lanes % 8)
        rv1 = rv0 + 2 * (_JU * 8)
        for k in range(_NBUF):
            g = ci * _NBUF + k
            b0 = k
            b1 = (k - 1) % _NBUF
            b2 = (k - 2) % _NBUF

            @pl.when(g == 0)
            def _prime():
                idx_copy(0, 0).start()

            @pl.when(g < _UPW)
            def _stage_a(g=g, b0=b0, b1=b1):
                @pl.when(g >= _NBUF)
                def _reclaim():
                    for cp in out_copies(g - _NBUF, b0):
                        cp.wait()
                idx_copy(g, b0).wait()
                for j in range(_JU):
                    tok_copy(b0, j).start()

            @pl.when(jnp.logical_and(g >= 1, g - 1 < _UPW))
            def _stage_b(b1=b1):
                for j in range(_JU):
                    tok_copy(b1, j).wait()
                    pltpu.async_copy(pos_hbm.at[idx_v.at[b1, j]],
                                     bufs[b1].at[pl.ds(j * 128, 128)],
                                     sem_g.at[b1], add=True)

            @pl.when(jnp.logical_and(g >= 2, g - 2 < _UPW))
            def _stage_c(g=g, b2=b2):
                for j in range(_JU):
                    pos_copy(b2, j).wait()

                def xpose(it, _c):
                    row0 = it * 8
                    j = lax.div(it, 16)
                    rj0 = rv0 + j * 8
                    rj1 = rv1 + j * 8
                    col = row0 - j * 128
                    for r in range(8):
                        v0 = bufs[b2][row0 + r, pl.ds(0, 16)]
                        v1 = bufs[b2][row0 + r, pl.ds(16, 16)]
                        cvec = jnp.full((16,), col + r, jnp.int32)
                        plsc.store_scatter(obufs[b2], [rj0, cvec], v0)
                        plsc.store_scatter(obufs[b2], [rj1, cvec], v1)
                    return 0

                lax.fori_loop(0, _ROWS // 8, xpose, 0)
                for cp in out_copies(g - 2, b2):
                    cp.start()

            @pl.when(jnp.logical_and(g + 1 >= 1, g + 1 < _UPW))
            def _prefetch(g=g, b2=b2):
                idx_copy(g + 1, b2).start()

        return 0

    lax.fori_loop(0, _NOUTER, step, 0)
    for k in range(_NBUF):
        gg = _UPW - _NBUF + k
        for cp in out_copies(gg, gg % _NBUF):
            cp.wait()


@jax.jit
def kernel(x, token_table, pos_table):
    xt = x.T.reshape(SEQ, _TB, 128)
    mesh = plsc.VectorSubcoreMesh(core_axis_name="c", subcore_axis_name="s")
    P = pl.kernel(
        _sc_body,
        out_type=jax.ShapeDtypeStruct((SEQ, EMBED // 8, _TB * 8, 128),
                                      jnp.float32),
        mesh=mesh,
        scratch_types=[
            pltpu.VMEM((_NBUF, _JU, 128), jnp.int32),
            pltpu.VMEM((_ROWS, EMBED), jnp.float32),
            pltpu.VMEM((_ROWS, EMBED), jnp.float32),
            pltpu.VMEM((_ROWS, EMBED), jnp.float32),
            pltpu.VMEM((4 * _JU * 8, _PB), jnp.float32),
            pltpu.VMEM((4 * _JU * 8, _PB), jnp.float32),
            pltpu.VMEM((4 * _JU * 8, _PB), jnp.float32),
            pltpu.SemaphoreType.DMA((_NBUF,)),
            pltpu.SemaphoreType.DMA((_NBUF,)),
            pltpu.SemaphoreType.DMA((_NBUF,)),
        ],
        compiler_params=pltpu.CompilerParams(use_tc_tiling_on_sc=False,
                                             needs_layout_passes=False),
    )(xt, token_table, pos_table)
    return (P.reshape(SEQ, EMBED // 8, _TB, 8, 128)
             .transpose(2, 4, 0, 1, 3).reshape(BATCH, SEQ, EMBED))


# speedup vs baseline: 1.6252x; 1.0499x over previous
import jax
import jax.numpy as jnp
from jax import lax
from jax.experimental import pallas as pl
from jax.experimental.pallas import tpu as pltpu
from jax.experimental.pallas import tpu_sc as plsc

VOCAB = 1000000
EMBED = 32
BATCH = 4096
SEQ = 200

_NC, _NS = 2, 16
_NW = _NC * _NS
_TB = BATCH // 128
_JU = 4
_QU = _TB // _JU
_NUNIT = SEQ * _QU
_UPW = _NUNIT // _NW
_ROWS = _JU * 128
_NBUF = 3
_NOUTER = (_UPW + 2 + _NBUF - 1) // _NBUF
_PB = 129


def _sc_body(xt_hbm, tok_hbm, pos_hbm, out_hbm,
             idx_v, buf0, buf1, buf2, ob0, ob1, ob2, sem_g, sem_w, sem_i):
    bufs = (buf0, buf1, buf2)
    obufs = (ob0, ob1, ob2)
    wid = lax.axis_index("s") * _NC + lax.axis_index("c")
    g0 = wid * _UPW

    def idx_copy(g, b):
        u = g0 + g
        s = lax.div(u, _QU)
        q = lax.rem(u, _QU)
        return pltpu.make_async_copy(
            xt_hbm.at[s, pl.ds(q * _JU, _JU)], idx_v.at[b], sem_i.at[b])

    def tok_copy(b, j):
        return pltpu.make_async_copy(
            tok_hbm.at[idx_v.at[b, j]],
            bufs[b].at[pl.ds(j * 128, 128)], sem_g.at[b])

    def pos_copy(b, j):
        return pltpu.make_async_copy(
            pos_hbm.at[idx_v.at[b, j]],
            bufs[b].at[pl.ds(j * 128, 128)], sem_g.at[b])

    def out_copies(g, b):
        u = g0 + g
        s = lax.div(u, _QU)
        q = lax.rem(u, _QU)
        return [pltpu.make_async_copy(
                    obufs[b].at[pl.ds(te * (_JU * 8), _JU * 8), pl.ds(0, 128)],
                    out_hbm.at[s, te, pl.ds(q * (_JU * 8), _JU * 8)],
                    sem_w.at[b])
                for te in range(4)]

    def step(ci, _):
        lanes = lax.iota(jnp.int32, 16)
        rv0 = (lanes // 8) * (_JU * 8) + (lanes % 8)
        rv1 = rv0 + 2 * (_JU * 8)
        for k in range(_NBUF):
            g = ci * _NBUF + k
            b0 = k
            b1 = (k - 1) % _NBUF
            b2 = (k - 2) % _NBUF

            @pl.when(g == 0)
            def _prime():
                idx_copy(0, 0).start()

            @pl.when(g < _UPW)
            def _stage_a(g=g, b0=b0, b1=b1):
                @pl.when(g >= _NBUF)
                def _reclaim():
                    for cp in out_copies(g - _NBUF, b0):
                        cp.wait()
                idx_copy(g, b0).wait()
                for j in range(_JU):
                    tok_copy(b0, j).start()

            @pl.when(jnp.logical_and(g >= 1, g - 1 < _UPW))
            def _stage_b(b1=b1):
                for j in range(_JU):
                    tok_copy(b1, j).wait()
                    pltpu.async_copy(pos_hbm.at[idx_v.at[b1, j]],
                                     bufs[b1].at[pl.ds(j * 128, 128)],
                                     sem_g.at[b1], add=True)

            @pl.when(jnp.logical_and(g >= 2, g - 2 < _UPW))
            def _stage_c(g=g, b2=b2):
                for j in range(_JU):
                    pos_copy(b2, j).wait()

                @plsc.parallel_loop(0, _ROWS // 8, unroll=2)
                def xpose(it):
                    row0 = it * 8
                    j = lax.div(it, 16)
                    rj0 = rv0 + j * 8
                    rj1 = rv1 + j * 8
                    cvec0 = jnp.full((16,), row0 - j * 128, jnp.int32)
                    for r in range(8):
                        v0 = bufs[b2][row0 + r, pl.ds(0, 16)]
                        v1 = bufs[b2][row0 + r, pl.ds(16, 16)]
                        cvec = cvec0 + r
                        plsc.store_scatter(obufs[b2], [rj0, cvec], v0)
                        plsc.store_scatter(obufs[b2], [rj1, cvec], v1)
                for cp in out_copies(g - 2, b2):
                    cp.start()

            @pl.when(jnp.logical_and(g + 1 >= 1, g + 1 < _UPW))
            def _prefetch(g=g, b2=b2):
                idx_copy(g + 1, b2).start()

        return 0

    lax.fori_loop(0, _NOUTER, step, 0)
    for k in range(_NBUF):
        gg = _UPW - _NBUF + k
        for cp in out_copies(gg, gg % _NBUF):
            cp.wait()


@jax.jit
def kernel(x, token_table, pos_table):
    xt = x.T.reshape(SEQ, _TB, 128)
    mesh = plsc.VectorSubcoreMesh(core_axis_name="c", subcore_axis_name="s")
    P = pl.kernel(
        _sc_body,
        out_type=jax.ShapeDtypeStruct((SEQ, EMBED // 8, _TB * 8, 128),
                                      jnp.float32),
        mesh=mesh,
        scratch_types=[
            pltpu.VMEM((_NBUF, _JU, 128), jnp.int32),
            pltpu.VMEM((_ROWS, EMBED), jnp.float32),
            pltpu.VMEM((_ROWS, EMBED), jnp.float32),
            pltpu.VMEM((_ROWS, EMBED), jnp.float32),
            pltpu.VMEM((4 * _JU * 8, _PB), jnp.float32),
            pltpu.VMEM((4 * _JU * 8, _PB), jnp.float32),
            pltpu.VMEM((4 * _JU * 8, _PB), jnp.float32),
            pltpu.SemaphoreType.DMA((_NBUF,)),
            pltpu.SemaphoreType.DMA((_NBUF,)),
            pltpu.SemaphoreType.DMA((_NBUF,)),
        ],
        compiler_params=pltpu.CompilerParams(use_tc_tiling_on_sc=False,
                                             needs_layout_passes=False),
    )(xt, token_table, pos_table)
    return (P.reshape(SEQ, EMBED // 8, _TB, 8, 128)
             .transpose(2, 4, 0, 1, 3).reshape(BATCH, SEQ, EMBED))
